# baseline (speedup 1.0000x reference)
import dataclasses

import jax
import jax.numpy as jnp
from jax import lax
from jax.experimental import pallas as pl
from jax.experimental.pallas import tpu as pltpu
from jax.experimental.pallas import tpu_sc as plsc

B = 16384
L = 50
D = 32
H = D // 2
V = 1000000

NUM_CORES = 2
NUM_SUBCORES = 16
NW = NUM_CORES * NUM_SUBCORES
RPW = B // NW
CHUNK = 64
NCHUNK = RPW // CHUNK
NIDX = CHUNK * L
XFER = 128
NXFER = NIDX // XFER
IDX_PAD = NIDX + 64

TCBLK = 512
NTCB = 999936 // TCBLK
XROWS = V // 4


def _tree_sum(xs):
    while len(xs) > 1:
        ys = [xs[i] + xs[i + 1] for i in range(0, len(xs) - 1, 2)]
        if len(xs) % 2:
            ys.append(xs[-1])
        xs = ys
    return xs[0]


def _tc_detrans_body(in_ref, out_ref):
    x = in_ref[...]
    r = lax.broadcasted_iota(jnp.int32, (TCBLK // 4, TCBLK), 0)
    c = lax.broadcasted_iota(jnp.int32, (TCBLK // 4, TCBLK), 1)
    outs = []
    for g in range(4):
        sel = jnp.where(c == 4 * r + g, jnp.float32(1.0),
                        jnp.float32(0.0))
        outs.append(lax.dot_general(
            sel, x, (((1,), (1,)), ((), ())),
            precision=lax.Precision.HIGHEST,
            preferred_element_type=jnp.float32))
    out_ref[...] = jnp.concatenate(outs, axis=1)


def _embed_mean_body(words_hbm, table_hbm, out_hbm, idx_v, rows_v, out_v,
                     t0_v, sem):
    wid = lax.axis_index("s") * NUM_CORES + lax.axis_index("c")
    pltpu.sync_copy(table_hbm.at[pl.ds(0, 1)], t0_v)
    t0_lo = t0_v[0, pl.ds(0, H)]
    t0_hi = t0_v[0, pl.ds(H, H)]
    lanes = lax.iota(jnp.int32, 16)
    scale = jnp.float32(1.0 / L)

    @pl.loop(0, NCHUNK)
    def _chunk(c):
        start = wid * (RPW * L) + c * NIDX
        pltpu.sync_copy(words_hbm.at[pl.ds(start, NIDX)],
                        idx_v.at[pl.ds(0, NIDX)])
        copies = [
            pltpu.async_copy(
                table_hbm.at[idx_v.at[pl.ds(j * XFER, XFER)]],
                rows_v.at[pl.ds(j * XFER, XFER)],
                sem,
            )
            for j in range(NXFER)
        ]
        for cp in copies:
            cp.wait()

        @pl.loop(0, CHUNK)
        def _row(i):
            base = i * L
            lo = [rows_v[base + j, pl.ds(0, H)] for j in range(L)]
            hi = [rows_v[base + j, pl.ds(H, H)] for j in range(L)]
            acc_lo = _tree_sum(lo)
            acc_hi = _tree_sum(hi)
            nz = jnp.float32(0.0)
            for q in range(4):
                pos = base + q * 16 + lanes
                if (q + 1) * 16 <= L:
                    vals = plsc.load_gather(idx_v, [pos])
                    hit = vals == 0
                else:
                    live = lanes < jnp.int32(L - q * 16)
                    vals = plsc.load_gather(idx_v, [pos], mask=live)
                    hit = jnp.logical_and(vals == 0, live)
                nz = nz + jnp.sum(jnp.where(hit, jnp.float32(1.0),
                                            jnp.float32(0.0)))
            out_v[i, pl.ds(0, H)] = (acc_lo - nz * t0_lo) * scale
            out_v[i, pl.ds(H, H)] = (acc_hi - nz * t0_hi) * scale

        pltpu.sync_copy(out_v,
                        out_hbm.at[pl.ds(wid * RPW + c * CHUNK, CHUNK)])


def kernel(words, table):
    x = pl.pallas_call(
        _tc_detrans_body,
        grid=(NTCB,),
        in_specs=[pl.BlockSpec((D, TCBLK), lambda i: (0, i))],
        out_specs=pl.BlockSpec((TCBLK // 4, 128), lambda i: (i, 0)),
        out_shape=jax.ShapeDtypeStruct((XROWS, 128), jnp.float32),
    )(table.T)
    tail = table[NTCB * TCBLK:].reshape(16, 128)
    x = lax.dynamic_update_slice(x, tail, (NTCB * TCBLK // 4, 0))
    table_lin = x.reshape(V, D)

    words_flat = words.reshape(B * L)
    mesh = plsc.VectorSubcoreMesh(core_axis_name="c", subcore_axis_name="s")
    cp = pltpu.CompilerParams(use_tc_tiling_on_sc=False)
    if "needs_layout_passes" in pltpu.CompilerParams.__dataclass_fields__:
        cp = dataclasses.replace(cp, needs_layout_passes=False)
    f = pl.kernel(
        _embed_mean_body,
        out_type=jax.ShapeDtypeStruct((B, D), jnp.float32),
        mesh=mesh,
        scratch_types=[
            pltpu.VMEM((IDX_PAD,), jnp.int32),
            pltpu.VMEM((NIDX, D), jnp.float32),
            pltpu.VMEM((CHUNK, D), jnp.float32),
            pltpu.VMEM((1, D), jnp.float32),
            pltpu.SemaphoreType.DMA,
        ],
        compiler_params=cp,
    )
    return f(words_flat, table_lin)


# speedup vs baseline: 3.5195x; 3.5195x over previous
import dataclasses

import jax
import jax.numpy as jnp
from jax import lax
from jax.experimental import pallas as pl
from jax.experimental.pallas import tpu as pltpu
from jax.experimental.pallas import tpu_sc as plsc

B = 16384
L = 50
D = 32
H = D // 2
V = 1000000

NUM_CORES = 2
NUM_SUBCORES = 16
NW = NUM_CORES * NUM_SUBCORES
RPW = B // NW
CHUNK = 64
NCHUNK = RPW // CHUNK
NIDX = CHUNK * L
XFER = 128
NXFER = NIDX // XFER
IDX_PAD = NIDX + 64

NBLK_FULL = V // 128
TAIL_W = V - NBLK_FULL * 128
BLK_PW = NBLK_FULL // NW
SS = 4
NSS = BLK_PW // SS
XROWS = V // 4


def _tree_sum(xs):
    while len(xs) > 1:
        ys = [xs[i] + xs[i + 1] for i in range(0, len(xs) - 1, 2)]
        if len(xs) % 2:
            ys.append(xs[-1])
        xs = ys
    return xs[0]


def _detrans_body(tt_hbm, x_hbm, in0, in1, out0, out1,
                  semi0, semi1, semo0, semo1):
    wid = lax.axis_index("s") * NUM_CORES + lax.axis_index("c")
    lanes = lax.iota(jnp.int32, 16)
    col0 = wid * BLK_PW * 128
    row0 = wid * BLK_PW * 32
    ins = (in0, in1)
    outs = (out0, out1)
    semis = (semi0, semi1)
    semos = (semo0, semo1)

    def fire_in(ss, par):
        pltpu.async_copy(tt_hbm.at[:, pl.ds(col0 + ss * (SS * 128),
                                            SS * 128)],
                         ins[par].at[:, pl.ds(0, SS * 128)], semis[par])

    def transpose_ss(ss, par):
        ib, ob = ins[par], outs[par]
        pltpu.make_async_copy(tt_hbm.at[:, pl.ds(0, SS * 128)],
                              ib.at[:, pl.ds(0, SS * 128)],
                              semis[par]).wait()

        @pl.loop(0, SS)
        def _blk(j):
            @pl.loop(0, 32, step=4)
            def _row(i):
                vals = []
                for r in range(4):
                    for s in range(8):
                        rows = 16 * (s % 2) + lanes
                        cols = jnp.full((16,), s // 2, jnp.int32) + (
                            j * 128 + (i + r) * 4)
                        vals.append(plsc.load_gather(ib, [rows, cols]))
                for r in range(4):
                    for s in range(8):
                        ob[j * 32 + i + r, pl.ds(s * 16, 16)] = (
                            vals[r * 8 + s])

        pltpu.async_copy(ob, x_hbm.at[pl.ds(row0 + ss * (SS * 32),
                                            SS * 32)], semos[par])

    fire_in(0, 0)

    @pl.loop(0, NSS + 1, step=2)
    def _steps(ss):
        for par in range(2):
            cur = ss + par

            @pl.when(cur < NSS)
            def _():
                @pl.when(cur + 1 < NSS)
                def _():
                    fire_in(cur + 1, (par + 1) % 2)

                @pl.when(cur >= 2)
                def _():
                    pltpu.make_async_copy(tt_hbm.at[:, pl.ds(0, SS * 128)],
                                          outs[par], semos[par]).wait()

                transpose_ss(cur, par)

    pltpu.make_async_copy(tt_hbm.at[:, pl.ds(0, SS * 128)], outs[0],
                          semos[0]).wait()
    pltpu.make_async_copy(tt_hbm.at[:, pl.ds(0, SS * 128)], outs[1],
                          semos[1]).wait()

    @pl.when(wid < 4)
    def _leftover():
        blk = NBLK_FULL - 4 + wid
        pltpu.sync_copy(tt_hbm.at[:, pl.ds(blk * 128, 128)],
                        in0.at[:, pl.ds(0, 128)])

        @pl.loop(0, 32)
        def _row(i):
            for s in range(8):
                rows = 16 * (s % 2) + lanes
                cols = jnp.full((16,), s // 2, jnp.int32) + i * 4
                out0[i, pl.ds(s * 16, 16)] = plsc.load_gather(
                    in0, [rows, cols])

        pltpu.sync_copy(out0.at[pl.ds(0, 32)],
                        x_hbm.at[pl.ds(blk * 32, 32)])


def _embed_mean_body(words_hbm, table_hbm, out_hbm, idx_v, rows_v, out_v,
                     t0_v, sem):
    wid = lax.axis_index("s") * NUM_CORES + lax.axis_index("c")
    pltpu.sync_copy(table_hbm.at[pl.ds(0, 1)], t0_v)
    t0_lo = t0_v[0, pl.ds(0, H)]
    t0_hi = t0_v[0, pl.ds(H, H)]
    lanes = lax.iota(jnp.int32, 16)
    scale = jnp.float32(1.0 / L)

    @pl.loop(0, NCHUNK)
    def _chunk(c):
        start = wid * (RPW * L) + c * NIDX
        pltpu.sync_copy(words_hbm.at[pl.ds(start, NIDX)],
                        idx_v.at[pl.ds(0, NIDX)])
        copies = [
            pltpu.async_copy(
                table_hbm.at[idx_v.at[pl.ds(j * XFER, XFER)]],
                rows_v.at[pl.ds(j * XFER, XFER)],
                sem,
            )
            for j in range(NXFER)
        ]
        for cp in copies:
            cp.wait()

        @pl.loop(0, CHUNK)
        def _row(i):
            base = i * L
            lo = [rows_v[base + j, pl.ds(0, H)] for j in range(L)]
            hi = [rows_v[base + j, pl.ds(H, H)] for j in range(L)]
            acc_lo = _tree_sum(lo)
            acc_hi = _tree_sum(hi)
            nz = jnp.float32(0.0)
            for q in range(4):
                pos = base + q * 16 + lanes
                if (q + 1) * 16 <= L:
                    vals = plsc.load_gather(idx_v, [pos])
                    hit = vals == 0
                else:
                    live = lanes < jnp.int32(L - q * 16)
                    vals = plsc.load_gather(idx_v, [pos], mask=live)
                    hit = jnp.logical_and(vals == 0, live)
                nz = nz + jnp.sum(jnp.where(hit, jnp.float32(1.0),
                                            jnp.float32(0.0)))
            out_v[i, pl.ds(0, H)] = (acc_lo - nz * t0_lo) * scale
            out_v[i, pl.ds(H, H)] = (acc_hi - nz * t0_hi) * scale

        pltpu.sync_copy(out_v,
                        out_hbm.at[pl.ds(wid * RPW + c * CHUNK, CHUNK)])


def kernel(words, table):
    mesh = plsc.VectorSubcoreMesh(core_axis_name="c", subcore_axis_name="s")

    cp_tiled = pltpu.CompilerParams(use_tc_tiling_on_sc=True,
                                    disable_bounds_checks=True)
    cp_lin = pltpu.CompilerParams(use_tc_tiling_on_sc=False,
                                  disable_bounds_checks=True)
    if "needs_layout_passes" in pltpu.CompilerParams.__dataclass_fields__:
        cp_tiled = dataclasses.replace(cp_tiled, needs_layout_passes=False)
        cp_lin = dataclasses.replace(cp_lin, needs_layout_passes=False)

    detrans = pl.kernel(
        _detrans_body,
        out_type=jax.ShapeDtypeStruct((XROWS, 128), jnp.float32),
        mesh=mesh,
        scratch_types=[
            pltpu.VMEM((D, SS * 128 + 1), jnp.float32),
            pltpu.VMEM((D, SS * 128 + 1), jnp.float32),
            pltpu.VMEM((SS * 32, 128), jnp.float32),
            pltpu.VMEM((SS * 32, 128), jnp.float32),
            pltpu.SemaphoreType.DMA,
            pltpu.SemaphoreType.DMA,
            pltpu.SemaphoreType.DMA,
            pltpu.SemaphoreType.DMA,
        ],
        compiler_params=cp_tiled,
    )
    x = detrans(table.T)
    tail = table[NBLK_FULL * 128:].reshape(TAIL_W // 4, 128)
    x = lax.dynamic_update_slice(x, tail, (NBLK_FULL * 32, 0))
    table_lin = x.reshape(V, D)

    words_flat = words.reshape(B * L)
    embed = pl.kernel(
        _embed_mean_body,
        out_type=jax.ShapeDtypeStruct((B, D), jnp.float32),
        mesh=mesh,
        scratch_types=[
            pltpu.VMEM((IDX_PAD,), jnp.int32),
            pltpu.VMEM((NIDX, D), jnp.float32),
            pltpu.VMEM((CHUNK, D), jnp.float32),
            pltpu.VMEM((1, D), jnp.float32),
            pltpu.SemaphoreType.DMA,
        ],
        compiler_params=cp_lin,
    )
    return embed(words_flat, table_lin)
